# baseline (speedup 1.0000x reference)
import functools

import jax
import jax.numpy as jnp
from jax import lax
from jax.experimental import pallas as pl
from jax.experimental.pallas import tpu as pltpu
from jax.experimental.pallas import tpu_sc as plsc

NC, NS = 2, 16
NT = NC * NS
WINE = 6400
OUT_TILE = 3128
ACC_PAD = 3136
UNROLL = 8


def _sc_segsum(x, ptrs, csr, bounds):
    n_nodes = x.shape[0]
    n_edges = ptrs.shape[0]
    assert n_edges % WINE == 0

    mesh = plsc.VectorSubcoreMesh(core_axis_name="c", subcore_axis_name="s")

    @functools.partial(
        pl.kernel,
        out_type=jax.ShapeDtypeStruct((n_nodes,), jnp.float32),
        mesh=mesh,
        compiler_params=pltpu.CompilerParams(needs_layout_passes=False),
        scratch_types=[
            pltpu.VMEM((n_nodes,), jnp.float32),
            pltpu.VMEM((ACC_PAD,), jnp.float32),
            [pltpu.VMEM((WINE,), jnp.int32)] * 2,
            [pltpu.VMEM((WINE + 32,), jnp.int32)] * 2,
            pltpu.VMEM((48,), jnp.int32),
            [pltpu.SemaphoreType.DMA] * 4,
        ],
    )
    def k(x_hbm, ptrs_hbm, csr_hbm, bounds_hbm, out_hbm,
          x_v, acc_v, pbufs, cbufs, bounds_v, sems):
        c = lax.axis_index("c")
        s = lax.axis_index("s")
        t = c * NS + s

        pltpu.sync_copy(x_hbm, x_v)
        pltpu.sync_copy(bounds_hbm, bounds_v)

        iota = lax.iota(jnp.int32, 16)
        zero16 = jnp.zeros((16,), jnp.float32)

        def zfill(i, carry):
            acc_v[pl.ds(i * 16, 16)] = zero16
            return carry

        lax.fori_loop(0, ACC_PAD // 16, zfill, 0)

        sent = jnp.full((16,), -1, jnp.int32)
        for slot in range(2):
            cbufs[slot][pl.ds(0, 16)] = sent
            cbufs[slot][pl.ds(WINE + 16, 16)] = sent

        lo = bounds_v[pl.ds(t, 16)][0]
        hi = bounds_v[pl.ds(t + 1, 16)][0]

        wlo = lo // WINE
        whi = (hi + WINE - 1) // WINE

        is15 = iota == 15
        shift_idx = {d: jnp.maximum(iota - d, 0) for d in (1, 2, 4, 8)}
        ge_mask = {d: iota >= d for d in (1, 2, 4, 8)}
        out_base = t * OUT_TILE

        grid_last = n_edges // WINE - 1

        def start_load(w, slot):
            pltpu.async_copy(ptrs_hbm.at[pl.ds(w * WINE, WINE)],
                             pbufs[slot], sems[2 * slot])
            pltpu.async_copy(csr_hbm.at[pl.ds(w * WINE, WINE)],
                             cbufs[slot].at[pl.ds(16, WINE)], sems[2 * slot + 1])

        def wait_load(slot):
            pltpu.make_async_copy(ptrs_hbm.at[pl.ds(0, WINE)],
                                  pbufs[slot], sems[2 * slot]).wait()
            pltpu.make_async_copy(csr_hbm.at[pl.ds(0, WINE)],
                                  cbufs[slot].at[pl.ds(16, WINE)],
                                  sems[2 * slot + 1]).wait()

        def compute(w, slot):
            pbuf = pbufs[slot]
            cbuf = cbufs[slot]
            ebase = w * WINE

            @plsc.parallel_loop(0, WINE // (16 * UNROLL))
            def grp(i):
                base = i * (16 * UNROLL)
                ps = [pbuf[pl.ds(base + u * 16, 16)] for u in range(UNROLL)]
                vs = [plsc.load_gather(x_v, [p]) for p in ps]
                ccs = [cbuf[pl.ds(16 + base + u * 16, 16)] for u in range(UNROLL)]
                prevs = [cbuf[pl.ds(15 + base + u * 16, 16)] for u in range(UNROLL)]
                nxts = [cbuf[pl.ds(17 + base + u * 16, 16)] for u in range(UNROLL)]
                pres = [plsc.cumsum(vs[u]) for u in range(UNROLL)]
                rss = [plsc.cummax(jnp.where(prevs[u] != ccs[u], iota, 0))
                       for u in range(UNROLL)]
                for u in range(UNROLL):
                    cc = ccs[u]
                    pre = pres[u]
                    rs = rss[u]
                    eidx = ebase + base + u * 16 + iota
                    own = (eidx >= lo) & (eidx < hi)
                    g = pre.at[jnp.maximum(rs - 1, 0)].get(
                        mode="promise_in_bounds")
                    run = pre - jnp.where(rs > 0, g, 0.0)
                    m = (is15 | (nxts[u] != cc)) & own
                    ci = jnp.clip(cc - out_base, 0, ACC_PAD - 1)
                    plsc.addupdate_scatter(acc_v, [ci], run, mask=m)

        nwin = whi - wlo
        npair = nwin // 2
        start_load(jnp.minimum(wlo, grid_last), 0)

        def pair(i, carry):
            w0 = wlo + 2 * i
            start_load(jnp.minimum(w0 + 1, grid_last), 1)
            wait_load(0)
            compute(w0, 0)
            start_load(jnp.minimum(w0 + 2, grid_last), 0)
            wait_load(1)
            compute(w0 + 1, 1)
            return carry

        lax.fori_loop(0, npair, pair, 0)
        wait_load(0)

        @pl.when(nwin - 2 * npair == 1)
        def _():
            compute(whi - 1, 0)

        @pl.when(t < NT - 1)
        def _():
            pltpu.sync_copy(acc_v.at[pl.ds(0, OUT_TILE)],
                            out_hbm.at[pl.ds(out_base, OUT_TILE)])

        @pl.when(t == NT - 1)
        def _():
            rem = n_nodes - (NT - 1) * OUT_TILE
            pltpu.sync_copy(acc_v.at[pl.ds(0, rem)],
                            out_hbm.at[pl.ds(out_base, rem)])

    return k(x, ptrs, csr, bounds)


def kernel(x, ptrs, csr):
    p1 = ptrs.astype(jnp.int32)
    c1 = csr.astype(jnp.int32)
    queries = jnp.arange(NT + 1, dtype=jnp.int32) * OUT_TILE
    bounds = jnp.searchsorted(c1, queries).astype(jnp.int32)
    bounds = jnp.pad(bounds, (0, 48 - (NT + 1)))
    return _sc_segsum(x.astype(jnp.float32), p1, c1, bounds)


# speedup vs baseline: 1.4261x; 1.4261x over previous
import functools

import jax
import jax.numpy as jnp
from jax import lax
from jax.experimental import pallas as pl
from jax.experimental.pallas import tpu as pltpu
from jax.experimental.pallas import tpu_sc as plsc

NC, NS = 2, 16
NT = NC * NS
WINE = 6400
OUT_TILE = 3128
ACC_PAD = 3136
UNROLL = 8


def _sc_segsum(x, ptrs, csr, bounds):
    n_nodes = x.shape[0]
    n_edges = ptrs.shape[0]
    assert n_edges % WINE == 0

    mesh = plsc.VectorSubcoreMesh(core_axis_name="c", subcore_axis_name="s")

    @functools.partial(
        pl.kernel,
        out_type=jax.ShapeDtypeStruct((n_nodes,), jnp.float32),
        mesh=mesh,
        compiler_params=pltpu.CompilerParams(needs_layout_passes=False),
        scratch_types=[
            pltpu.VMEM((n_nodes,), jnp.float32),
            pltpu.VMEM((ACC_PAD,), jnp.float32),
            [pltpu.VMEM((WINE,), jnp.int32)] * 2,
            [pltpu.VMEM((WINE + 32,), jnp.int32)] * 2,
            pltpu.VMEM((48,), jnp.int32),
            [pltpu.SemaphoreType.DMA] * 4,
        ],
    )
    def k(x_hbm, ptrs_hbm, csr_hbm, bounds_hbm, out_hbm,
          x_v, acc_v, pbufs, cbufs, bounds_v, sems):
        c = lax.axis_index("c")
        s = lax.axis_index("s")
        t = c * NS + s

        pltpu.sync_copy(x_hbm, x_v)
        pltpu.sync_copy(bounds_hbm, bounds_v)

        iota = lax.iota(jnp.int32, 16)
        zero16 = jnp.zeros((16,), jnp.float32)

        def zfill(i, carry):
            acc_v[pl.ds(i * 16, 16)] = zero16
            return carry

        lax.fori_loop(0, ACC_PAD // 16, zfill, 0)

        sent = jnp.full((16,), -1, jnp.int32)
        for slot in range(2):
            cbufs[slot][pl.ds(0, 16)] = sent
            cbufs[slot][pl.ds(WINE + 16, 16)] = sent

        lo = bounds_v[pl.ds(t, 16)][0]
        hi = bounds_v[pl.ds(t + 1, 16)][0]

        wlo = lo // WINE
        whi = (hi + WINE - 1) // WINE

        is15 = iota == 15
        shift_idx = {d: jnp.maximum(iota - d, 0) for d in (1, 2, 4, 8)}
        ge_mask = {d: iota >= d for d in (1, 2, 4, 8)}
        out_base = t * OUT_TILE

        grid_last = n_edges // WINE - 1

        def start_load(w, slot):
            pltpu.async_copy(ptrs_hbm.at[pl.ds(w * WINE, WINE)],
                             pbufs[slot], sems[2 * slot])
            pltpu.async_copy(csr_hbm.at[pl.ds(w * WINE, WINE)],
                             cbufs[slot].at[pl.ds(16, WINE)], sems[2 * slot + 1])

        def wait_load(slot):
            pltpu.make_async_copy(ptrs_hbm.at[pl.ds(0, WINE)],
                                  pbufs[slot], sems[2 * slot]).wait()
            pltpu.make_async_copy(csr_hbm.at[pl.ds(0, WINE)],
                                  cbufs[slot].at[pl.ds(16, WINE)],
                                  sems[2 * slot + 1]).wait()

        def compute(w, slot, masked):
            pbuf = pbufs[slot]
            cbuf = cbufs[slot]
            ebase = w * WINE

            def grp(i, carry2):
                base = i * (16 * UNROLL)
                ps = [pbuf[pl.ds(base + u * 16, 16)] for u in range(UNROLL)]
                vs = [plsc.load_gather(x_v, [p]) for p in ps]
                ccs = [cbuf[pl.ds(16 + base + u * 16, 16)] for u in range(UNROLL)]
                prevs = [cbuf[pl.ds(15 + base + u * 16, 16)] for u in range(UNROLL)]
                nxts = [cbuf[pl.ds(17 + base + u * 16, 16)] for u in range(UNROLL)]
                pres = [plsc.cumsum(vs[u]) for u in range(UNROLL)]
                rss = [plsc.cummax(jnp.where(prevs[u] != ccs[u], iota, 0))
                       for u in range(UNROLL)]
                for u in range(UNROLL):
                    cc = ccs[u]
                    pre = pres[u]
                    rs = rss[u]
                    g = pre.at[jnp.maximum(rs - 1, 0)].get(
                        mode="promise_in_bounds")
                    run = pre - jnp.where(rs > 0, g, 0.0)
                    m = is15 | (nxts[u] != cc)
                    if masked:
                        eidx = ebase + base + u * 16 + iota
                        m = m & (eidx >= lo) & (eidx < hi)
                        ci = jnp.clip(cc - out_base, 0, ACC_PAD - 1)
                    else:
                        ci = cc - out_base
                    plsc.addupdate_scatter(acc_v, [ci], run, mask=m)
                return carry2

            lax.fori_loop(0, WINE // (16 * UNROLL), grp, 0)

        wflo = (lo + WINE - 1) // WINE
        wfhi = hi // WINE
        head_end = jnp.minimum(whi, wflo)
        tail_start = jnp.maximum(wfhi, head_end)

        @pl.when(wlo < head_end)
        def _():
            start_load(wlo, 0)
            wait_load(0)
            compute(wlo, 0, True)

        @pl.when(tail_start < whi)
        def _():
            start_load(whi - 1, 0)
            wait_load(0)
            compute(whi - 1, 0, True)

        nwin = jnp.maximum(wfhi - wflo, 0)
        npair = nwin // 2
        start_load(jnp.minimum(wflo, grid_last), 0)

        def pair(i, carry):
            w0 = wflo + 2 * i
            start_load(jnp.minimum(w0 + 1, grid_last), 1)
            wait_load(0)
            compute(w0, 0, False)
            start_load(jnp.minimum(w0 + 2, grid_last), 0)
            wait_load(1)
            compute(w0 + 1, 1, False)
            return carry

        lax.fori_loop(0, npair, pair, 0)
        wait_load(0)

        @pl.when(nwin - 2 * npair == 1)
        def _():
            compute(wfhi - 1, 0, False)

        @pl.when(t < NT - 1)
        def _():
            pltpu.sync_copy(acc_v.at[pl.ds(0, OUT_TILE)],
                            out_hbm.at[pl.ds(out_base, OUT_TILE)])

        @pl.when(t == NT - 1)
        def _():
            rem = n_nodes - (NT - 1) * OUT_TILE
            pltpu.sync_copy(acc_v.at[pl.ds(0, rem)],
                            out_hbm.at[pl.ds(out_base, rem)])

    return k(x, ptrs, csr, bounds)


def kernel(x, ptrs, csr):
    p1 = ptrs.astype(jnp.int32)
    c1 = csr.astype(jnp.int32)
    queries = jnp.arange(NT + 1, dtype=jnp.int32) * OUT_TILE
    bounds = jnp.searchsorted(c1, queries).astype(jnp.int32)
    bounds = jnp.pad(bounds, (0, 48 - (NT + 1)))
    return _sc_segsum(x.astype(jnp.float32), p1, c1, bounds)


# speedup vs baseline: 1.4600x; 1.0237x over previous
import functools

import jax
import jax.numpy as jnp
from jax import lax
from jax.experimental import pallas as pl
from jax.experimental.pallas import tpu as pltpu
from jax.experimental.pallas import tpu_sc as plsc

NC, NS = 2, 16
NT = NC * NS
WINE = 6400
OUT_TILE = 3128
ACC_PAD = 3136
UNROLL = 10


def _sc_segsum(x, ptrs, csr, bounds):
    n_nodes = x.shape[0]
    n_edges = ptrs.shape[0]
    assert n_edges % WINE == 0

    mesh = plsc.VectorSubcoreMesh(core_axis_name="c", subcore_axis_name="s")

    @functools.partial(
        pl.kernel,
        out_type=jax.ShapeDtypeStruct((n_nodes,), jnp.float32),
        mesh=mesh,
        compiler_params=pltpu.CompilerParams(needs_layout_passes=False),
        scratch_types=[
            pltpu.VMEM((n_nodes,), jnp.float32),
            pltpu.VMEM((ACC_PAD,), jnp.float32),
            [pltpu.VMEM((WINE,), jnp.int32)] * 2,
            [pltpu.VMEM((WINE + 32,), jnp.int32)] * 2,
            pltpu.VMEM((48,), jnp.int32),
            [pltpu.SemaphoreType.DMA] * 4,
        ],
    )
    def k(x_hbm, ptrs_hbm, csr_hbm, bounds_hbm, out_hbm,
          x_v, acc_v, pbufs, cbufs, bounds_v, sems):
        c = lax.axis_index("c")
        s = lax.axis_index("s")
        t = c * NS + s

        pltpu.sync_copy(x_hbm, x_v)
        pltpu.sync_copy(bounds_hbm, bounds_v)

        iota = lax.iota(jnp.int32, 16)
        zero16 = jnp.zeros((16,), jnp.float32)

        def zfill(i, carry):
            acc_v[pl.ds(i * 16, 16)] = zero16
            return carry

        lax.fori_loop(0, ACC_PAD // 16, zfill, 0)

        sent = jnp.full((16,), -1, jnp.int32)
        for slot in range(2):
            cbufs[slot][pl.ds(0, 16)] = sent
            cbufs[slot][pl.ds(WINE + 16, 16)] = sent

        lo = bounds_v[pl.ds(t, 16)][0]
        hi = bounds_v[pl.ds(t + 1, 16)][0]

        wlo = lo // WINE
        whi = (hi + WINE - 1) // WINE

        is15 = iota == 15
        shift_idx = {d: jnp.maximum(iota - d, 0) for d in (1, 2, 4, 8)}
        ge_mask = {d: iota >= d for d in (1, 2, 4, 8)}
        out_base = t * OUT_TILE

        grid_last = n_edges // WINE - 1

        def start_load(w, slot):
            pltpu.async_copy(ptrs_hbm.at[pl.ds(w * WINE, WINE)],
                             pbufs[slot], sems[2 * slot])
            pltpu.async_copy(csr_hbm.at[pl.ds(w * WINE, WINE)],
                             cbufs[slot].at[pl.ds(16, WINE)], sems[2 * slot + 1])

        def wait_load(slot):
            pltpu.make_async_copy(ptrs_hbm.at[pl.ds(0, WINE)],
                                  pbufs[slot], sems[2 * slot]).wait()
            pltpu.make_async_copy(csr_hbm.at[pl.ds(0, WINE)],
                                  cbufs[slot].at[pl.ds(16, WINE)],
                                  sems[2 * slot + 1]).wait()

        def compute(w, slot, masked):
            pbuf = pbufs[slot]
            cbuf = cbufs[slot]
            ebase = w * WINE

            def grp(i, carry2):
                base = i * (16 * UNROLL)
                ps = [pbuf[pl.ds(base + u * 16, 16)] for u in range(UNROLL)]
                vs = [plsc.load_gather(x_v, [p]) for p in ps]
                ccs = [cbuf[pl.ds(16 + base + u * 16, 16)] for u in range(UNROLL)]
                prevs = [cbuf[pl.ds(15 + base + u * 16, 16)] for u in range(UNROLL)]
                nxts = [cbuf[pl.ds(17 + base + u * 16, 16)] for u in range(UNROLL)]
                pres = [plsc.cumsum(vs[u]) for u in range(UNROLL)]
                rss = [plsc.cummax(jnp.where(prevs[u] != ccs[u], iota, 0))
                       for u in range(UNROLL)]
                for u in range(UNROLL):
                    cc = ccs[u]
                    pre = pres[u]
                    rs = rss[u]
                    g = pre.at[jnp.maximum(rs - 1, 0)].get(
                        mode="promise_in_bounds")
                    run = pre - jnp.where(rs > 0, g, 0.0)
                    m = is15 | (nxts[u] != cc)
                    if masked:
                        eidx = ebase + base + u * 16 + iota
                        m = m & (eidx >= lo) & (eidx < hi)
                        ci = jnp.clip(cc - out_base, 0, ACC_PAD - 1)
                    else:
                        ci = cc - out_base
                    plsc.addupdate_scatter(acc_v, [ci], run, mask=m)
                return carry2

            lax.fori_loop(0, WINE // (16 * UNROLL), grp, 0)

        wflo = (lo + WINE - 1) // WINE
        wfhi = hi // WINE
        head_end = jnp.minimum(whi, wflo)
        tail_start = jnp.maximum(wfhi, head_end)

        @pl.when(wlo < head_end)
        def _():
            start_load(wlo, 0)
            wait_load(0)
            compute(wlo, 0, True)

        @pl.when(tail_start < whi)
        def _():
            start_load(whi - 1, 0)
            wait_load(0)
            compute(whi - 1, 0, True)

        nwin = jnp.maximum(wfhi - wflo, 0)
        npair = nwin // 2
        start_load(jnp.minimum(wflo, grid_last), 0)

        def pair(i, carry):
            w0 = wflo + 2 * i
            start_load(jnp.minimum(w0 + 1, grid_last), 1)
            wait_load(0)
            compute(w0, 0, False)
            start_load(jnp.minimum(w0 + 2, grid_last), 0)
            wait_load(1)
            compute(w0 + 1, 1, False)
            return carry

        lax.fori_loop(0, npair, pair, 0)
        wait_load(0)

        @pl.when(nwin - 2 * npair == 1)
        def _():
            compute(wfhi - 1, 0, False)

        @pl.when(t < NT - 1)
        def _():
            pltpu.sync_copy(acc_v.at[pl.ds(0, OUT_TILE)],
                            out_hbm.at[pl.ds(out_base, OUT_TILE)])

        @pl.when(t == NT - 1)
        def _():
            rem = n_nodes - (NT - 1) * OUT_TILE
            pltpu.sync_copy(acc_v.at[pl.ds(0, rem)],
                            out_hbm.at[pl.ds(out_base, rem)])

    return k(x, ptrs, csr, bounds)


def kernel(x, ptrs, csr):
    p1 = ptrs.astype(jnp.int32)
    c1 = csr.astype(jnp.int32)
    queries = jnp.arange(NT + 1, dtype=jnp.int32) * OUT_TILE
    bounds = jnp.searchsorted(c1, queries).astype(jnp.int32)
    bounds = jnp.pad(bounds, (0, 48 - (NT + 1)))
    return _sc_segsum(x.astype(jnp.float32), p1, c1, bounds)


# speedup vs baseline: 1.4771x; 1.0118x over previous
import functools

import jax
import jax.numpy as jnp
from jax import lax
from jax.experimental import pallas as pl
from jax.experimental.pallas import tpu as pltpu
from jax.experimental.pallas import tpu_sc as plsc

NC, NS = 2, 16
NT = NC * NS
WINE = 6400
OUT_TILE = 3128
ACC_PAD = 3136
UNROLL = 16


def _sc_segsum(x, ptrs, csr, bounds):
    n_nodes = x.shape[0]
    n_edges = ptrs.shape[0]
    assert n_edges % WINE == 0

    mesh = plsc.VectorSubcoreMesh(core_axis_name="c", subcore_axis_name="s")

    @functools.partial(
        pl.kernel,
        out_type=jax.ShapeDtypeStruct((n_nodes,), jnp.float32),
        mesh=mesh,
        compiler_params=pltpu.CompilerParams(needs_layout_passes=False),
        scratch_types=[
            pltpu.VMEM((n_nodes,), jnp.float32),
            pltpu.VMEM((ACC_PAD,), jnp.float32),
            [pltpu.VMEM((WINE,), jnp.int32)] * 2,
            [pltpu.VMEM((WINE + 32,), jnp.int32)] * 2,
            pltpu.VMEM((48,), jnp.int32),
            [pltpu.SemaphoreType.DMA] * 4,
        ],
    )
    def k(x_hbm, ptrs_hbm, csr_hbm, bounds_hbm, out_hbm,
          x_v, acc_v, pbufs, cbufs, bounds_v, sems):
        c = lax.axis_index("c")
        s = lax.axis_index("s")
        t = c * NS + s

        pltpu.sync_copy(x_hbm, x_v)
        pltpu.sync_copy(bounds_hbm, bounds_v)

        iota = lax.iota(jnp.int32, 16)
        zero16 = jnp.zeros((16,), jnp.float32)

        def zfill(i, carry):
            acc_v[pl.ds(i * 16, 16)] = zero16
            return carry

        lax.fori_loop(0, ACC_PAD // 16, zfill, 0)

        sent = jnp.full((16,), -1, jnp.int32)
        for slot in range(2):
            cbufs[slot][pl.ds(0, 16)] = sent
            cbufs[slot][pl.ds(WINE + 16, 16)] = sent

        lo = bounds_v[pl.ds(t, 16)][0]
        hi = bounds_v[pl.ds(t + 1, 16)][0]

        wlo = lo // WINE
        whi = (hi + WINE - 1) // WINE

        is15 = iota == 15
        shift_idx = {d: jnp.maximum(iota - d, 0) for d in (1, 2, 4, 8)}
        ge_mask = {d: iota >= d for d in (1, 2, 4, 8)}
        out_base = t * OUT_TILE

        grid_last = n_edges // WINE - 1

        def start_load(w, slot):
            pltpu.async_copy(ptrs_hbm.at[pl.ds(w * WINE, WINE)],
                             pbufs[slot], sems[2 * slot])
            pltpu.async_copy(csr_hbm.at[pl.ds(w * WINE, WINE)],
                             cbufs[slot].at[pl.ds(16, WINE)], sems[2 * slot + 1])

        def wait_load(slot):
            pltpu.make_async_copy(ptrs_hbm.at[pl.ds(0, WINE)],
                                  pbufs[slot], sems[2 * slot]).wait()
            pltpu.make_async_copy(csr_hbm.at[pl.ds(0, WINE)],
                                  cbufs[slot].at[pl.ds(16, WINE)],
                                  sems[2 * slot + 1]).wait()

        def compute(w, slot, masked):
            pbuf = pbufs[slot]
            cbuf = cbufs[slot]
            ebase = w * WINE

            def grp(i, carry2):
                base = i * (16 * UNROLL)
                ps = [pbuf[pl.ds(base + u * 16, 16)] for u in range(UNROLL)]
                vs = [plsc.load_gather(x_v, [p]) for p in ps]
                ccs = [cbuf[pl.ds(16 + base + u * 16, 16)] for u in range(UNROLL)]
                prevs = [cbuf[pl.ds(15 + base + u * 16, 16)] for u in range(UNROLL)]
                nxts = [cbuf[pl.ds(17 + base + u * 16, 16)] for u in range(UNROLL)]
                pres = [plsc.cumsum(vs[u]) for u in range(UNROLL)]
                rss = [plsc.cummax(jnp.where(prevs[u] != ccs[u], iota, 0))
                       for u in range(UNROLL)]
                for u in range(UNROLL):
                    cc = ccs[u]
                    pre = pres[u]
                    rs = rss[u]
                    g = pre.at[jnp.maximum(rs - 1, 0)].get(
                        mode="promise_in_bounds")
                    run = pre - jnp.where(rs > 0, g, 0.0)
                    m = is15 | (nxts[u] != cc)
                    if masked:
                        eidx = ebase + base + u * 16 + iota
                        m = m & (eidx >= lo) & (eidx < hi)
                        ci = jnp.clip(cc - out_base, 0, ACC_PAD - 1)
                    else:
                        ci = cc - out_base
                    plsc.addupdate_scatter(acc_v, [ci], run, mask=m)
                return carry2

            lax.fori_loop(0, WINE // (16 * UNROLL), grp, 0)

        wflo = (lo + WINE - 1) // WINE
        wfhi = hi // WINE
        head_end = jnp.minimum(whi, wflo)
        tail_start = jnp.maximum(wfhi, head_end)

        @pl.when(wlo < head_end)
        def _():
            start_load(wlo, 0)
            wait_load(0)
            compute(wlo, 0, True)

        @pl.when(tail_start < whi)
        def _():
            start_load(whi - 1, 0)
            wait_load(0)
            compute(whi - 1, 0, True)

        nwin = jnp.maximum(wfhi - wflo, 0)
        npair = nwin // 2
        start_load(jnp.minimum(wflo, grid_last), 0)

        def pair(i, carry):
            w0 = wflo + 2 * i
            start_load(jnp.minimum(w0 + 1, grid_last), 1)
            wait_load(0)
            compute(w0, 0, False)
            start_load(jnp.minimum(w0 + 2, grid_last), 0)
            wait_load(1)
            compute(w0 + 1, 1, False)
            return carry

        lax.fori_loop(0, npair, pair, 0)
        wait_load(0)

        @pl.when(nwin - 2 * npair == 1)
        def _():
            compute(wfhi - 1, 0, False)

        @pl.when(t < NT - 1)
        def _():
            pltpu.sync_copy(acc_v.at[pl.ds(0, OUT_TILE)],
                            out_hbm.at[pl.ds(out_base, OUT_TILE)])

        @pl.when(t == NT - 1)
        def _():
            rem = n_nodes - (NT - 1) * OUT_TILE
            pltpu.sync_copy(acc_v.at[pl.ds(0, rem)],
                            out_hbm.at[pl.ds(out_base, rem)])

    return k(x, ptrs, csr, bounds)


def kernel(x, ptrs, csr):
    p1 = ptrs.astype(jnp.int32)
    c1 = csr.astype(jnp.int32)
    queries = jnp.arange(NT + 1, dtype=jnp.int32) * OUT_TILE
    bounds = jnp.searchsorted(c1, queries).astype(jnp.int32)
    bounds = jnp.pad(bounds, (0, 48 - (NT + 1)))
    return _sc_segsum(x.astype(jnp.float32), p1, c1, bounds)


# speedup vs baseline: 1.4839x; 1.0046x over previous
import functools

import jax
import jax.numpy as jnp
from jax import lax
from jax.experimental import pallas as pl
from jax.experimental.pallas import tpu as pltpu
from jax.experimental.pallas import tpu_sc as plsc

NC, NS = 2, 16
NT = NC * NS
WINE = 6400
OUT_TILE = 3128
ACC_PAD = 3136
UNROLL = 20


def _sc_segsum(x, ptrs, csr, bounds):
    n_nodes = x.shape[0]
    n_edges = ptrs.shape[0]
    assert n_edges % WINE == 0

    mesh = plsc.VectorSubcoreMesh(core_axis_name="c", subcore_axis_name="s")

    @functools.partial(
        pl.kernel,
        out_type=jax.ShapeDtypeStruct((n_nodes,), jnp.float32),
        mesh=mesh,
        compiler_params=pltpu.CompilerParams(needs_layout_passes=False),
        scratch_types=[
            pltpu.VMEM((n_nodes,), jnp.float32),
            pltpu.VMEM((ACC_PAD,), jnp.float32),
            [pltpu.VMEM((WINE,), jnp.int32)] * 2,
            [pltpu.VMEM((WINE + 32,), jnp.int32)] * 2,
            pltpu.VMEM((48,), jnp.int32),
            [pltpu.SemaphoreType.DMA] * 4,
        ],
    )
    def k(x_hbm, ptrs_hbm, csr_hbm, bounds_hbm, out_hbm,
          x_v, acc_v, pbufs, cbufs, bounds_v, sems):
        c = lax.axis_index("c")
        s = lax.axis_index("s")
        t = c * NS + s

        pltpu.sync_copy(x_hbm, x_v)
        pltpu.sync_copy(bounds_hbm, bounds_v)

        iota = lax.iota(jnp.int32, 16)
        zero16 = jnp.zeros((16,), jnp.float32)

        def zfill(i, carry):
            acc_v[pl.ds(i * 16, 16)] = zero16
            return carry

        lax.fori_loop(0, ACC_PAD // 16, zfill, 0)

        sent = jnp.full((16,), -1, jnp.int32)
        for slot in range(2):
            cbufs[slot][pl.ds(0, 16)] = sent
            cbufs[slot][pl.ds(WINE + 16, 16)] = sent

        lo = bounds_v[pl.ds(t, 16)][0]
        hi = bounds_v[pl.ds(t + 1, 16)][0]

        wlo = lo // WINE
        whi = (hi + WINE - 1) // WINE

        is15 = iota == 15
        shift_idx = {d: jnp.maximum(iota - d, 0) for d in (1, 2, 4, 8)}
        ge_mask = {d: iota >= d for d in (1, 2, 4, 8)}
        out_base = t * OUT_TILE

        grid_last = n_edges // WINE - 1

        def start_load(w, slot):
            pltpu.async_copy(ptrs_hbm.at[pl.ds(w * WINE, WINE)],
                             pbufs[slot], sems[2 * slot])
            pltpu.async_copy(csr_hbm.at[pl.ds(w * WINE, WINE)],
                             cbufs[slot].at[pl.ds(16, WINE)], sems[2 * slot + 1])

        def wait_load(slot):
            pltpu.make_async_copy(ptrs_hbm.at[pl.ds(0, WINE)],
                                  pbufs[slot], sems[2 * slot]).wait()
            pltpu.make_async_copy(csr_hbm.at[pl.ds(0, WINE)],
                                  cbufs[slot].at[pl.ds(16, WINE)],
                                  sems[2 * slot + 1]).wait()

        def compute(w, slot, masked):
            pbuf = pbufs[slot]
            cbuf = cbufs[slot]
            ebase = w * WINE

            def grp(i, carry2):
                base = i * (16 * UNROLL)
                ps = [pbuf[pl.ds(base + u * 16, 16)] for u in range(UNROLL)]
                vs = [plsc.load_gather(x_v, [p]) for p in ps]
                ccs = [cbuf[pl.ds(16 + base + u * 16, 16)] for u in range(UNROLL)]
                prevs = [cbuf[pl.ds(15 + base + u * 16, 16)] for u in range(UNROLL)]
                nxts = [cbuf[pl.ds(17 + base + u * 16, 16)] for u in range(UNROLL)]
                pres = [plsc.cumsum(vs[u]) for u in range(UNROLL)]
                rss = [plsc.cummax(jnp.where(prevs[u] != ccs[u], iota, 0))
                       for u in range(UNROLL)]
                for u in range(UNROLL):
                    cc = ccs[u]
                    pre = pres[u]
                    rs = rss[u]
                    g = pre.at[jnp.maximum(rs - 1, 0)].get(
                        mode="promise_in_bounds")
                    run = pre - jnp.where(rs > 0, g, 0.0)
                    m = is15 | (nxts[u] != cc)
                    if masked:
                        eidx = ebase + base + u * 16 + iota
                        m = m & (eidx >= lo) & (eidx < hi)
                        ci = jnp.clip(cc - out_base, 0, ACC_PAD - 1)
                    else:
                        ci = cc - out_base
                    plsc.addupdate_scatter(acc_v, [ci], run, mask=m)
                return carry2

            lax.fori_loop(0, WINE // (16 * UNROLL), grp, 0)

        wflo = (lo + WINE - 1) // WINE
        wfhi = hi // WINE
        head_end = jnp.minimum(whi, wflo)
        tail_start = jnp.maximum(wfhi, head_end)

        @pl.when(wlo < head_end)
        def _():
            start_load(wlo, 0)
            wait_load(0)
            compute(wlo, 0, True)

        @pl.when(tail_start < whi)
        def _():
            start_load(whi - 1, 0)
            wait_load(0)
            compute(whi - 1, 0, True)

        nwin = jnp.maximum(wfhi - wflo, 0)
        npair = nwin // 2
        start_load(jnp.minimum(wflo, grid_last), 0)

        def pair(i, carry):
            w0 = wflo + 2 * i
            start_load(jnp.minimum(w0 + 1, grid_last), 1)
            wait_load(0)
            compute(w0, 0, False)
            start_load(jnp.minimum(w0 + 2, grid_last), 0)
            wait_load(1)
            compute(w0 + 1, 1, False)
            return carry

        lax.fori_loop(0, npair, pair, 0)
        wait_load(0)

        @pl.when(nwin - 2 * npair == 1)
        def _():
            compute(wfhi - 1, 0, False)

        @pl.when(t < NT - 1)
        def _():
            pltpu.sync_copy(acc_v.at[pl.ds(0, OUT_TILE)],
                            out_hbm.at[pl.ds(out_base, OUT_TILE)])

        @pl.when(t == NT - 1)
        def _():
            rem = n_nodes - (NT - 1) * OUT_TILE
            pltpu.sync_copy(acc_v.at[pl.ds(0, rem)],
                            out_hbm.at[pl.ds(out_base, rem)])

    return k(x, ptrs, csr, bounds)


def kernel(x, ptrs, csr):
    p1 = ptrs.astype(jnp.int32)
    c1 = csr.astype(jnp.int32)
    queries = jnp.arange(NT + 1, dtype=jnp.int32) * OUT_TILE
    bounds = jnp.searchsorted(c1, queries).astype(jnp.int32)
    bounds = jnp.pad(bounds, (0, 48 - (NT + 1)))
    return _sc_segsum(x.astype(jnp.float32), p1, c1, bounds)
